# baseline (speedup 1.0000x reference)
import functools
import jax
import jax.numpy as jnp
from jax import lax
from jax.experimental import pallas as pl
from jax.experimental.pallas import tpu as pltpu
from jax.experimental.pallas import tpu_sc as plsc

_SC_CORES = 2
_SC_SUBCORES = 16
_SC_WORKERS = _SC_CORES * _SC_SUBCORES
_NBUF = 4


def _sc_copy(emg):
    B, T, F = emg.shape
    bpw = B // _SC_WORKERS
    mesh = plsc.VectorSubcoreMesh(core_axis_name="c", subcore_axis_name="s")

    @functools.partial(
        pl.kernel,
        mesh=mesh,
        out_type=jax.ShapeDtypeStruct((B, T, F), jnp.float32),
        scratch_types=[
            pltpu.VMEM((_NBUF, T, F), jnp.float32),
            pltpu.SemaphoreType.DMA,
            pltpu.SemaphoreType.DMA,
        ],
    )
    def copy_k(src_hbm, dst_hbm, buf, s_in, s_out):
        wid = lax.axis_index("s") * _SC_CORES + lax.axis_index("c")
        base = wid * bpw

        def group(g, carry):
            j0 = base + g * _NBUF
            for k in range(_NBUF):
                pltpu.make_async_copy(
                    src_hbm.at[j0 + k], buf.at[k], s_in
                ).start()
            for k in range(_NBUF):
                pltpu.make_async_copy(
                    src_hbm.at[j0 + k], buf.at[k], s_in
                ).wait()
            for k in range(_NBUF):
                pltpu.make_async_copy(
                    buf.at[k], dst_hbm.at[j0 + k], s_out
                ).start()
            for k in range(_NBUF):
                pltpu.make_async_copy(
                    buf.at[k], dst_hbm.at[j0 + k], s_out
                ).wait()
            return carry

        lax.fori_loop(0, bpw // _NBUF, group, 0)

    return copy_k(emg)


def kernel(emg_features, session_ids, table):
    B, T, F = emg_features.shape
    copied = _sc_copy(emg_features)
    embed = jnp.take(table, session_ids.astype(jnp.int32), axis=0)
    embed = jnp.broadcast_to(embed[:, None, :], (B, T, embed.shape[-1]))
    return jnp.concatenate([copied, embed], axis=-1)


# speedup vs baseline: 1.3149x; 1.3149x over previous
import functools
import jax
import jax.numpy as jnp
from jax import lax
from jax.experimental import pallas as pl
from jax.experimental.pallas import tpu as pltpu
from jax.experimental.pallas import tpu_sc as plsc

_BB = 8

_SC_CORES = 2
_SC_SUBCORES = 16
_SC_WORKERS = _SC_CORES * _SC_SUBCORES


def _sc_gather(table, sids):
    B = sids.shape[0]
    N, E = table.shape
    b_per_w = B // _SC_WORKERS
    mesh = plsc.VectorSubcoreMesh(core_axis_name="c", subcore_axis_name="s")

    @functools.partial(
        pl.kernel,
        mesh=mesh,
        out_type=jax.ShapeDtypeStruct((B, E), jnp.float32),
        scratch_types=[
            pltpu.VMEM((b_per_w,), jnp.int32),
            pltpu.VMEM((b_per_w, E), jnp.float32),
            pltpu.SemaphoreType.DMA,
        ],
    )
    def gather_k(table_hbm, idx_hbm, out_hbm, idx_v, rows_v, sem):
        wid = lax.axis_index("s") * _SC_CORES + lax.axis_index("c")
        base = wid * b_per_w
        pltpu.sync_copy(idx_hbm.at[pl.ds(base, b_per_w)], idx_v)
        pltpu.async_copy(table_hbm.at[idx_v], rows_v, sem).wait()
        pltpu.sync_copy(rows_v, out_hbm.at[pl.ds(base, b_per_w)])

    return gather_k(table, sids)


def _concat_body(emg_ref, emb_ref, out_ref, *, T):
    F = emg_ref.shape[-1]
    E = out_ref.shape[-1] - F
    out_ref[:, :F] = emg_ref[...]
    for j in range(_BB):
        row = emb_ref[j, :E]
        out_ref[pl.ds(j * T, T), F:] = jnp.broadcast_to(row[None, :], (T, E))


def _tc_concat(emg_features, embed):
    B, T, F = emg_features.shape
    E = 144 - F
    emg2d = jnp.reshape(emg_features, (B * T, F))
    out2d = pl.pallas_call(
        functools.partial(_concat_body, T=T),
        grid=(B // _BB,),
        in_specs=[
            pl.BlockSpec((_BB * T, F), lambda i: (i, 0)),
            pl.BlockSpec((_BB, embed.shape[-1]), lambda i: (i, 0)),
        ],
        out_specs=pl.BlockSpec((_BB * T, F + E), lambda i: (i, 0)),
        out_shape=jax.ShapeDtypeStruct((B * T, F + E), jnp.float32),
    )(emg2d, embed)
    return jnp.reshape(out2d, (B, T, F + E))


def kernel(emg_features, session_ids, table):
    sids = session_ids.astype(jnp.int32)
    table_p = jnp.pad(table, ((0, 0), (0, 128 - table.shape[1])))
    embed = _sc_gather(table_p, sids)
    return _tc_concat(emg_features, embed)


# speedup vs baseline: 1.4516x; 1.1040x over previous
import functools
import jax
import jax.numpy as jnp
from jax import lax
from jax.experimental import pallas as pl
from jax.experimental.pallas import tpu as pltpu
from jax.experimental.pallas import tpu_sc as plsc

_BB = 8

_SC_CORES = 2
_SC_SUBCORES = 16
_SC_WORKERS = _SC_CORES * _SC_SUBCORES


def _sc_gather(table, sids):
    B = sids.shape[0]
    N, E = table.shape
    b_per_w = B // _SC_WORKERS
    mesh = plsc.VectorSubcoreMesh(core_axis_name="c", subcore_axis_name="s")

    @functools.partial(
        pl.kernel,
        mesh=mesh,
        out_type=jax.ShapeDtypeStruct((B, E), jnp.float32),
        scratch_types=[
            pltpu.VMEM((b_per_w,), jnp.int32),
            pltpu.VMEM((b_per_w, E), jnp.float32),
            pltpu.SemaphoreType.DMA,
        ],
    )
    def gather_k(table_hbm, idx_hbm, out_hbm, idx_v, rows_v, sem):
        wid = lax.axis_index("s") * _SC_CORES + lax.axis_index("c")
        base = wid * b_per_w
        pltpu.sync_copy(idx_hbm.at[pl.ds(base, b_per_w)], idx_v)
        pltpu.async_copy(table_hbm.at[idx_v], rows_v, sem).wait()
        pltpu.sync_copy(rows_v, out_hbm.at[pl.ds(base, b_per_w)])

    return gather_k(table, sids)


def _concat_body(emg_ref, emb_ref, out_ref, *, T, F, E):
    W = F + E
    emb = emb_ref[:, :E]
    for t in range(T):
        out_ref[:, t * W : t * W + F] = emg_ref[:, t * F : (t + 1) * F]
        out_ref[:, t * W + F : (t + 1) * W] = emb


def _tc_concat(emg_features, embed):
    B, T, F = emg_features.shape
    E = 144 - F
    emg2d = jnp.reshape(emg_features, (B, T * F))
    out2d = pl.pallas_call(
        functools.partial(_concat_body, T=T, F=F, E=E),
        grid=(B // _BB,),
        in_specs=[
            pl.BlockSpec((_BB, T * F), lambda i: (i, 0)),
            pl.BlockSpec((_BB, embed.shape[-1]), lambda i: (i, 0)),
        ],
        out_specs=pl.BlockSpec((_BB, T * (F + E)), lambda i: (i, 0)),
        out_shape=jax.ShapeDtypeStruct((B, T * (F + E)), jnp.float32),
    )(emg2d, embed)
    return jnp.reshape(out2d, (B, T, F + E))


def kernel(emg_features, session_ids, table):
    sids = session_ids.astype(jnp.int32)
    table_p = jnp.pad(table, ((0, 0), (0, 128 - table.shape[1])))
    embed = _sc_gather(table_p, sids)
    return _tc_concat(emg_features, embed)


# speedup vs baseline: 1.6177x; 1.1144x over previous
import functools
import jax
import jax.numpy as jnp
from jax import lax
from jax.experimental import pallas as pl
from jax.experimental.pallas import tpu as pltpu
from jax.experimental.pallas import tpu_sc as plsc

_BB = 16

_SC_CORES = 2
_SC_SUBCORES = 16
_SC_WORKERS = _SC_CORES * _SC_SUBCORES


def _sc_gather(table, sids):
    B = sids.shape[0]
    N, E = table.shape
    b_per_w = B // _SC_WORKERS
    mesh = plsc.VectorSubcoreMesh(core_axis_name="c", subcore_axis_name="s")

    @functools.partial(
        pl.kernel,
        mesh=mesh,
        out_type=jax.ShapeDtypeStruct((B, E), jnp.float32),
        scratch_types=[
            pltpu.VMEM((b_per_w,), jnp.int32),
            pltpu.VMEM((b_per_w, E), jnp.float32),
            pltpu.SemaphoreType.DMA,
        ],
    )
    def gather_k(table_hbm, idx_hbm, out_hbm, idx_v, rows_v, sem):
        wid = lax.axis_index("s") * _SC_CORES + lax.axis_index("c")
        base = wid * b_per_w
        pltpu.sync_copy(idx_hbm.at[pl.ds(base, b_per_w)], idx_v)
        pltpu.async_copy(table_hbm.at[idx_v], rows_v, sem).wait()
        pltpu.sync_copy(rows_v, out_hbm.at[pl.ds(base, b_per_w)])

    return gather_k(table, sids)


def _concat_body(emg_ref, emb_ref, out_ref, *, T, F, E):
    W = F + E
    emb = emb_ref[:, :E]
    for t in range(T):
        out_ref[:, t * W : t * W + F] = emg_ref[:, t * F : (t + 1) * F]
        out_ref[:, t * W + F : (t + 1) * W] = emb


def _tc_concat(emg_features, embed):
    B, T, F = emg_features.shape
    E = 144 - F
    emg2d = jnp.reshape(emg_features, (B, T * F))
    out2d = pl.pallas_call(
        functools.partial(_concat_body, T=T, F=F, E=E),
        grid=(B // _BB,),
        in_specs=[
            pl.BlockSpec((_BB, T * F), lambda i: (i, 0)),
            pl.BlockSpec((_BB, embed.shape[-1]), lambda i: (i, 0)),
        ],
        out_specs=pl.BlockSpec((_BB, T * (F + E)), lambda i: (i, 0)),
        out_shape=jax.ShapeDtypeStruct((B, T * (F + E)), jnp.float32),
    )(emg2d, embed)
    return jnp.reshape(out2d, (B, T, F + E))


def kernel(emg_features, session_ids, table):
    sids = session_ids.astype(jnp.int32)
    table_p = jnp.pad(table, ((0, 0), (0, 128 - table.shape[1])))
    embed = _sc_gather(table_p, sids)
    return _tc_concat(emg_features, embed)


# speedup vs baseline: 1.7251x; 1.0664x over previous
import functools
import jax
import jax.numpy as jnp
from jax import lax
from jax.experimental import pallas as pl
from jax.experimental.pallas import tpu as pltpu
from jax.experimental.pallas import tpu_sc as plsc

_BB = 32

_SC_CORES = 2
_SC_SUBCORES = 16
_SC_WORKERS = _SC_CORES * _SC_SUBCORES


def _sc_gather(table, sids):
    B = sids.shape[0]
    N, E = table.shape
    b_per_w = B // _SC_WORKERS
    mesh = plsc.VectorSubcoreMesh(core_axis_name="c", subcore_axis_name="s")

    @functools.partial(
        pl.kernel,
        mesh=mesh,
        out_type=jax.ShapeDtypeStruct((B, E), jnp.float32),
        scratch_types=[
            pltpu.VMEM((b_per_w,), jnp.int32),
            pltpu.VMEM((b_per_w, E), jnp.float32),
            pltpu.SemaphoreType.DMA,
        ],
    )
    def gather_k(table_hbm, idx_hbm, out_hbm, idx_v, rows_v, sem):
        wid = lax.axis_index("s") * _SC_CORES + lax.axis_index("c")
        base = wid * b_per_w
        pltpu.sync_copy(idx_hbm.at[pl.ds(base, b_per_w)], idx_v)
        pltpu.async_copy(table_hbm.at[idx_v], rows_v, sem).wait()
        pltpu.sync_copy(rows_v, out_hbm.at[pl.ds(base, b_per_w)])

    return gather_k(table, sids)


def _concat_body(emg_ref, emb_ref, out_ref, *, T, F, E):
    W = F + E
    emb = emb_ref[:, :E]
    for t in range(T):
        out_ref[:, t * W : t * W + F] = emg_ref[:, t * F : (t + 1) * F]
        out_ref[:, t * W + F : (t + 1) * W] = emb


def _tc_concat(emg_features, embed):
    B, T, F = emg_features.shape
    E = 144 - F
    emg2d = jnp.reshape(emg_features, (B, T * F))
    out2d = pl.pallas_call(
        functools.partial(_concat_body, T=T, F=F, E=E),
        grid=(B // _BB,),
        in_specs=[
            pl.BlockSpec((_BB, T * F), lambda i: (i, 0)),
            pl.BlockSpec((_BB, embed.shape[-1]), lambda i: (i, 0)),
        ],
        out_specs=pl.BlockSpec((_BB, T * (F + E)), lambda i: (i, 0)),
        out_shape=jax.ShapeDtypeStruct((B, T * (F + E)), jnp.float32),
    )(emg2d, embed)
    return jnp.reshape(out2d, (B, T, F + E))


def kernel(emg_features, session_ids, table):
    sids = session_ids.astype(jnp.int32)
    table_p = jnp.pad(table, ((0, 0), (0, 128 - table.shape[1])))
    embed = _sc_gather(table_p, sids)
    return _tc_concat(emg_features, embed)


# speedup vs baseline: 1.7578x; 1.0189x over previous
import functools
import jax
import jax.numpy as jnp
from jax import lax
from jax.experimental import pallas as pl
from jax.experimental.pallas import tpu as pltpu
from jax.experimental.pallas import tpu_sc as plsc

_BB = 64

_SC_CORES = 2
_SC_SUBCORES = 16
_SC_WORKERS = _SC_CORES * _SC_SUBCORES


def _sc_gather(table, sids):
    B = sids.shape[0]
    N, E = table.shape
    b_per_w = B // _SC_WORKERS
    mesh = plsc.VectorSubcoreMesh(core_axis_name="c", subcore_axis_name="s")

    @functools.partial(
        pl.kernel,
        mesh=mesh,
        out_type=jax.ShapeDtypeStruct((B, E), jnp.float32),
        scratch_types=[
            pltpu.VMEM((b_per_w,), jnp.int32),
            pltpu.VMEM((b_per_w, E), jnp.float32),
            pltpu.SemaphoreType.DMA,
        ],
    )
    def gather_k(table_hbm, idx_hbm, out_hbm, idx_v, rows_v, sem):
        wid = lax.axis_index("s") * _SC_CORES + lax.axis_index("c")
        base = wid * b_per_w
        pltpu.sync_copy(idx_hbm.at[pl.ds(base, b_per_w)], idx_v)
        pltpu.async_copy(table_hbm.at[idx_v], rows_v, sem).wait()
        pltpu.sync_copy(rows_v, out_hbm.at[pl.ds(base, b_per_w)])

    return gather_k(table, sids)


def _concat_body(emg_ref, emb_ref, out_ref, *, T, F, E):
    W = F + E
    emb = emb_ref[:, :E]
    for t in range(T):
        out_ref[:, t * W : t * W + F] = emg_ref[:, t * F : (t + 1) * F]
        out_ref[:, t * W + F : (t + 1) * W] = emb


def _tc_concat(emg_features, embed):
    B, T, F = emg_features.shape
    E = 144 - F
    emg2d = jnp.reshape(emg_features, (B, T * F))
    out2d = pl.pallas_call(
        functools.partial(_concat_body, T=T, F=F, E=E),
        grid=(B // _BB,),
        in_specs=[
            pl.BlockSpec((_BB, T * F), lambda i: (i, 0)),
            pl.BlockSpec((_BB, embed.shape[-1]), lambda i: (i, 0)),
        ],
        out_specs=pl.BlockSpec((_BB, T * (F + E)), lambda i: (i, 0)),
        out_shape=jax.ShapeDtypeStruct((B, T * (F + E)), jnp.float32),
    )(emg2d, embed)
    return jnp.reshape(out2d, (B, T, F + E))


def kernel(emg_features, session_ids, table):
    sids = session_ids.astype(jnp.int32)
    table_p = jnp.pad(table, ((0, 0), (0, 128 - table.shape[1])))
    embed = _sc_gather(table_p, sids)
    return _tc_concat(emg_features, embed)


# speedup vs baseline: 1.7586x; 1.0004x over previous
import functools
import jax
import jax.numpy as jnp
from jax import lax
from jax.experimental import pallas as pl
from jax.experimental.pallas import tpu as pltpu
from jax.experimental.pallas import tpu_sc as plsc

_BB = 128

_SC_CORES = 2
_SC_SUBCORES = 16
_SC_WORKERS = _SC_CORES * _SC_SUBCORES


def _sc_gather(table, sids):
    B = sids.shape[0]
    N, E = table.shape
    b_per_w = B // _SC_WORKERS
    mesh = plsc.VectorSubcoreMesh(core_axis_name="c", subcore_axis_name="s")

    @functools.partial(
        pl.kernel,
        mesh=mesh,
        out_type=jax.ShapeDtypeStruct((B, E), jnp.float32),
        scratch_types=[
            pltpu.VMEM((b_per_w,), jnp.int32),
            pltpu.VMEM((b_per_w, E), jnp.float32),
            pltpu.SemaphoreType.DMA,
        ],
    )
    def gather_k(table_hbm, idx_hbm, out_hbm, idx_v, rows_v, sem):
        wid = lax.axis_index("s") * _SC_CORES + lax.axis_index("c")
        base = wid * b_per_w
        pltpu.sync_copy(idx_hbm.at[pl.ds(base, b_per_w)], idx_v)
        pltpu.async_copy(table_hbm.at[idx_v], rows_v, sem).wait()
        pltpu.sync_copy(rows_v, out_hbm.at[pl.ds(base, b_per_w)])

    return gather_k(table, sids)


def _concat_body(emg_ref, emb_ref, out_ref, *, T, F, E):
    W = F + E
    emb = emb_ref[:, :E]
    for t in range(T):
        out_ref[:, t * W : t * W + F] = emg_ref[:, t * F : (t + 1) * F]
        out_ref[:, t * W + F : (t + 1) * W] = emb


def _tc_concat(emg_features, embed):
    B, T, F = emg_features.shape
    E = 144 - F
    emg2d = jnp.reshape(emg_features, (B, T * F))
    out2d = pl.pallas_call(
        functools.partial(_concat_body, T=T, F=F, E=E),
        grid=(B // _BB,),
        in_specs=[
            pl.BlockSpec((_BB, T * F), lambda i: (i, 0)),
            pl.BlockSpec((_BB, embed.shape[-1]), lambda i: (i, 0)),
        ],
        out_specs=pl.BlockSpec((_BB, T * (F + E)), lambda i: (i, 0)),
        out_shape=jax.ShapeDtypeStruct((B, T * (F + E)), jnp.float32),
        compiler_params=pltpu.CompilerParams(
            vmem_limit_bytes=128 * 1024 * 1024,
        ),
    )(emg2d, embed)
    return jnp.reshape(out2d, (B, T, F + E))


def kernel(emg_features, session_ids, table):
    sids = session_ids.astype(jnp.int32)
    table_p = jnp.pad(table, ((0, 0), (0, 128 - table.shape[1])))
    embed = _sc_gather(table_p, sids)
    return _tc_concat(emg_features, embed)


# speedup vs baseline: 1.7596x; 1.0006x over previous
import functools
import jax
import jax.numpy as jnp
from jax import lax
from jax.experimental import pallas as pl
from jax.experimental.pallas import tpu as pltpu
from jax.experimental.pallas import tpu_sc as plsc

_BB = 64

_SC_CORES = 2
_SC_SUBCORES = 16
_SC_WORKERS = _SC_CORES * _SC_SUBCORES


def _sc_gather(table, sids):
    B = sids.shape[0]
    N, E = table.shape
    b_per_w = B // _SC_WORKERS
    mesh = plsc.VectorSubcoreMesh(core_axis_name="c", subcore_axis_name="s")

    @functools.partial(
        pl.kernel,
        mesh=mesh,
        out_type=jax.ShapeDtypeStruct((B, E), jnp.float32),
        scratch_types=[
            pltpu.VMEM((b_per_w,), jnp.int32),
            pltpu.VMEM((b_per_w, E), jnp.float32),
            pltpu.SemaphoreType.DMA,
        ],
    )
    def gather_k(table_hbm, idx_hbm, out_hbm, idx_v, rows_v, sem):
        wid = lax.axis_index("s") * _SC_CORES + lax.axis_index("c")
        base = wid * b_per_w
        pltpu.sync_copy(idx_hbm.at[pl.ds(base, b_per_w)], idx_v)
        pltpu.async_copy(table_hbm.at[idx_v], rows_v, sem).wait()
        pltpu.sync_copy(rows_v, out_hbm.at[pl.ds(base, b_per_w)])

    return gather_k(table, sids)


def _concat_body(emg_ref, emb_ref, out_ref, *, T, F, E):
    W = F + E
    emb = emb_ref[:, :E]
    for t in range(T):
        out_ref[:, t * W : t * W + F] = emg_ref[:, t * F : (t + 1) * F]
        out_ref[:, t * W + F : (t + 1) * W] = emb


def _tc_concat(emg_features, embed, E):
    B, T, F = emg_features.shape
    emg2d = jnp.reshape(emg_features, (B, T * F))
    out2d = pl.pallas_call(
        functools.partial(_concat_body, T=T, F=F, E=E),
        grid=(B // _BB,),
        in_specs=[
            pl.BlockSpec((_BB, T * F), lambda i: (i, 0)),
            pl.BlockSpec((_BB, embed.shape[-1]), lambda i: (i, 0)),
        ],
        out_specs=pl.BlockSpec((_BB, T * (F + E)), lambda i: (i, 0)),
        out_shape=jax.ShapeDtypeStruct((B, T * (F + E)), jnp.float32),
    )(emg2d, embed)
    return jnp.reshape(out2d, (B, T, F + E))


def kernel(emg_features, session_ids, table):
    sids = session_ids.astype(jnp.int32)
    table_p = jnp.pad(table, ((0, 0), (0, 128 - table.shape[1])))
    embed = _sc_gather(table_p, sids)
    return _tc_concat(emg_features, embed, table.shape[1])
